# baseline (speedup 1.0000x reference)
import functools

import jax
import jax.numpy as jnp
from jax import lax
from jax.experimental import pallas as pl
from jax.experimental.pallas import tpu as pltpu
from jax.experimental.pallas import tpu_sc as plsc

_INFO = plsc.get_sparse_core_info()
_NC = _INFO.num_cores
_NS = _INFO.num_subcores
_NW = _NC * _NS


@functools.partial(jax.jit, static_argnames=("chunk",))
def _embed(x, table, chunk=4):
    R, C = x.shape
    V, D = table.shape
    r_per_w = R // _NW
    n_chunks = r_per_w // chunk
    n_pairs = n_chunks // 2
    mesh = plsc.VectorSubcoreMesh(core_axis_name="c", subcore_axis_name="s")

    @functools.partial(
        pl.kernel,
        mesh=mesh,
        out_type=jax.ShapeDtypeStruct((R, C, D), jnp.float32),
        scratch_types=[
            pltpu.VMEM((2, chunk, C), jnp.int32),
            pltpu.VMEM((2, chunk, C, D), jnp.float32),
            pltpu.SemaphoreType.DMA,
            pltpu.SemaphoreType.DMA,
            pltpu.SemaphoreType.DMA,
            pltpu.SemaphoreType.DMA,
        ],
        compiler_params=pltpu.CompilerParams(use_tc_tiling_on_sc=False),
    )
    def k(table_hbm, idx_hbm, out_hbm, idx_v, rows_v, g0, g1, w0, w1):
        wid = lax.axis_index("s") * _NC + lax.axis_index("c")
        w_base = wid * r_per_w
        gsem = (g0, g1)
        wsem = (w0, w1)

        def idx_src(g):
            return idx_hbm.at[pl.ds(w_base + g * chunk, chunk)]

        def out_dst(g):
            return out_hbm.at[pl.ds(w_base + g * chunk, chunk)]

        def gather(b):
            for j in range(chunk):
                pltpu.async_copy(
                    table_hbm.at[idx_v.at[b].at[j]],
                    rows_v.at[b].at[j], gsem[b])

        def gather_wait(b):
            for j in range(chunk):
                pltpu.make_async_copy(
                    table_hbm.at[idx_v.at[b].at[j]],
                    rows_v.at[b].at[j], gsem[b]).wait()

        def write(g, b):
            pltpu.async_copy(rows_v.at[b], out_dst(g), wsem[b])

        def write_wait(g, b):
            pltpu.make_async_copy(rows_v.at[b], out_dst(g), wsem[b]).wait()

        pltpu.sync_copy(idx_src(0), idx_v.at[0])
        gather(0)

        def pair_body(p, _):
            for b in (0, 1):
                g = 2 * p + b
                nb = 1 - b
                if b == 1:
                    write_wait(g - 1, nb)
                else:
                    @pl.when(p > 0)
                    def _():
                        write_wait(g - 1, nb)
                if b == 0:
                    pltpu.sync_copy(idx_src(g + 1), idx_v.at[nb])
                    gather(nb)
                else:
                    @pl.when(p < n_pairs - 1)
                    def _():
                        pltpu.sync_copy(idx_src(g + 1), idx_v.at[nb])
                        gather(nb)
                gather_wait(b)
                write(g, b)
            return 0

        lax.fori_loop(0, n_pairs, pair_body, 0)
        write_wait(n_chunks - 1, 1)

    return k(table, x)


def kernel(x, table):
    return _embed(x.astype(jnp.int32), table)


# speedup vs baseline: 1.2794x; 1.2794x over previous
import functools

import jax
import jax.numpy as jnp
from jax import lax
from jax.experimental import pallas as pl
from jax.experimental.pallas import tpu as pltpu
from jax.experimental.pallas import tpu_sc as plsc

_INFO = plsc.get_sparse_core_info()
_NC = _INFO.num_cores
_NS = _INFO.num_subcores
_NW = _NC * _NS

_LP = 133


def _embed(x, table):
    R, C = x.shape
    V, D = table.shape
    TR = D // 8
    TC = R // 128
    i_per_w = R // _NW
    CH = i_per_w // 2
    n_units = C * 2
    mesh = plsc.VectorSubcoreMesh(core_axis_name="c", subcore_axis_name="s")

    xt = x.T

    @functools.partial(
        pl.kernel,
        mesh=mesh,
        out_type=jax.ShapeDtypeStruct((C, TR, TC, 8, 128), jnp.float32),
        scratch_types=[
            pltpu.VMEM((2, CH), jnp.int32),
            pltpu.VMEM((2, CH, D), jnp.float32),
            pltpu.VMEM((2, 2 * TR, 8, _LP), jnp.float32),
            pltpu.SemaphoreType.DMA,
            pltpu.SemaphoreType.DMA,
            pltpu.SemaphoreType.DMA,
            pltpu.SemaphoreType.DMA,
        ],
        compiler_params=pltpu.CompilerParams(
            use_tc_tiling_on_sc=False, needs_layout_passes=False),
    )
    def k(table_hbm, xt_hbm, out_hbm, idx_v, rows_v, t5p, g0, g1, w0, w1):
        wid = lax.axis_index("s") * _NC + lax.axis_index("c")
        i0w = wid * i_per_w
        tc0 = wid * (i_per_w // 128)
        gsem = (g0, g1)
        wsem = (w0, w1)

        iota = lax.iota(jnp.int32, 16)
        c_trdiv = iota >> 3
        c_s = iota & 7

        def idx_src(u):
            j, h = u >> 1, u & 1
            return xt_hbm.at[j, pl.ds(i0w + h * CH, CH)]

        def gather(b):
            pltpu.async_copy(table_hbm.at[idx_v.at[b]], rows_v.at[b], gsem[b])

        def gather_wait(b):
            pltpu.make_async_copy(
                table_hbm.at[idx_v.at[b]], rows_v.at[b], gsem[b]).wait()

        def out_dst(u, tcx):
            j, h = u >> 1, u & 1
            return out_hbm.at[j, :, tc0 + 2 * h + tcx]

        def t5_src(b, tcx):
            return t5p.at[b, pl.ds(8 * tcx, 8), :, pl.ds(0, 128)]

        def write(u, b):
            for tcx in (0, 1):
                pltpu.async_copy(t5_src(b, tcx), out_dst(u, tcx), wsem[b])

        def write_wait(u, b):
            for tcx in (0, 1):
                pltpu.make_async_copy(
                    t5_src(b, tcx), out_dst(u, tcx), wsem[b]).wait()

        def transpose(b):
            def rr_body(rr, _):
                r0 = rr * 32
                for ri in range(32):
                    r = r0 + ri
                    tcx8 = (r >> 7) << 3
                    l_vec = jnp.full((16,), r & 127, jnp.int32)
                    for d0 in (0, 16, 32, 48):
                        v = rows_v[b, r, pl.ds(d0, 16)]
                        trr = c_trdiv + (tcx8 + (d0 >> 3))
                        plsc.store_scatter(
                            t5p.at[b], [trr, c_s, l_vec], v)
                return 0
            lax.fori_loop(0, CH // 32, rr_body, 0)

        pltpu.sync_copy(idx_src(0), idx_v.at[0])
        gather(0)

        def pair_body(p, _):
            for b in (0, 1):
                u = 2 * p + b
                nb = 1 - b
                gather_wait(b)
                if b == 0:
                    pltpu.sync_copy(idx_src(u + 1), idx_v.at[nb])
                    gather(nb)
                else:
                    @pl.when(p < C - 1)
                    def _():
                        pltpu.sync_copy(idx_src(u + 1), idx_v.at[nb])
                        gather(nb)
                @pl.when(p > 0)
                def _():
                    write_wait(u - 2, b)
                transpose(b)
                write(u, b)
            return 0

        lax.fori_loop(0, C, pair_body, 0)
        write_wait(n_units - 2, 0)
        write_wait(n_units - 1, 1)

    out5 = k(table, xt)
    return out5.transpose(2, 4, 0, 1, 3).reshape(R, C, D)


def kernel(x, table):
    return _embed(x.astype(jnp.int32), table)
